# baseline (speedup 1.0000x reference)
import jax
import jax.numpy as jnp
from jax.experimental import pallas as pl


def _copy3(h_ref, c_ref, o1_ref, o2_ref, o3_ref):
    o1_ref[...] = h_ref[...]
    o2_ref[...] = c_ref[...]
    o3_ref[...] = h_ref[...] * 2.0


def kernel(x, edge_index, edge_weight, h, c, W_i, W_f, W_c, W_o, conv_i_w,
           conv_i_b, conv_f_w, conv_f_b, conv_c_w, conv_c_b, conv_o_w,
           conv_o_b, w_c_i, w_c_f, w_c_o, b_i, b_f, b_c, b_o, lin_w, lin_b):
    del edge_index, edge_weight
    hh, cc, _h2 = pl.pallas_call(
        _copy3,
        out_shape=[
            jax.ShapeDtypeStruct(h.shape, h.dtype),
            jax.ShapeDtypeStruct(c.shape, c.dtype),
            jax.ShapeDtypeStruct(h.shape, h.dtype),
        ],
    )(h, c)
    I = jax.nn.sigmoid(x @ W_i + hh @ conv_i_w + conv_i_b + w_c_i * cc + b_i)
    Fg = jax.nn.sigmoid(x @ W_f + hh @ conv_f_w + conv_f_b + w_c_f * cc + b_f)
    T = jnp.tanh(x @ W_c + hh @ conv_c_w + conv_c_b + b_c)
    C = Fg * cc + I * T
    O = jax.nn.sigmoid(x @ W_o + hh @ conv_o_w + conv_o_b + w_c_o * C + b_o)
    H = O * jnp.tanh(C)
    out = H @ lin_w + lin_b + 0.0 * _h2[:, :1]
    return (out, H, C)


# speedup vs baseline: 1.2031x; 1.2031x over previous
import jax
import jax.numpy as jnp
from jax.experimental import pallas as pl
from jax.experimental.pallas import tpu as pltpu

F_OUT = 32


def _gclstm_block(x_hbm, h_hbm, c_hbm, wi_ref, wf_ref, wc_ref, wo_ref,
                  ci_ref, cf_ref, cc_ref, co_ref, cib_ref, cfb_ref, ccb_ref,
                  cob_ref, wci_ref, wcf_ref, wco_ref, bi_ref, bf_ref, bc_ref,
                  bo_ref, linw_ref, linb_ref, out_hbm, h_out_hbm, c_out_hbm,
                  x_vm, h_vm, c_vm, out_vm, h_new_vm, c_new_vm,
                  s0, s1, s2, s3, s4, s5):
    f4 = 4 * F_OUT
    cp_in = [pltpu.make_async_copy(x_hbm, x_vm, s0),
             pltpu.make_async_copy(h_hbm, h_vm, s1),
             pltpu.make_async_copy(c_hbm, c_vm, s2)]
    for cpy in cp_in:
        cpy.start()
    for cpy in cp_in:
        cpy.wait()
    wp = jnp.concatenate(
        [wi_ref[...], wf_ref[...], wc_ref[...], wo_ref[...]], axis=1)
    cp = jnp.concatenate(
        [ci_ref[...], cf_ref[...], cc_ref[...], co_ref[...]], axis=1)
    x4 = x_vm[...]
    h4 = h_vm[...]
    c4 = c_vm[...]
    gj = [jnp.dot(x4[:, 128 * j:128 * (j + 1)], wp,
                  preferred_element_type=jnp.float32)
          + jnp.dot(h4[:, F_OUT * j:F_OUT * (j + 1)], cp,
                    preferred_element_type=jnp.float32)
          for j in range(4)]
    g4 = jnp.concatenate(
        [gj[j][:, F_OUT * g:F_OUT * (g + 1)] for g in range(4)
         for j in range(4)], axis=1)
    bias4 = jnp.concatenate(
        [jnp.concatenate([blk] * 4, axis=1) for blk in
         (cib_ref[...] + bi_ref[...], cfb_ref[...] + bf_ref[...],
          ccb_ref[...] + bc_ref[...], cob_ref[...] + bo_ref[...])], axis=1)
    g4 = g4 + bias4
    wci4 = jnp.concatenate([wci_ref[...]] * 4, axis=1)
    wcf4 = jnp.concatenate([wcf_ref[...]] * 4, axis=1)
    wco4 = jnp.concatenate([wco_ref[...]] * 4, axis=1)
    i_g = jax.nn.sigmoid(g4[:, 0 * f4:1 * f4] + wci4 * c4)
    f_g = jax.nn.sigmoid(g4[:, 1 * f4:2 * f4] + wcf4 * c4)
    t_g = jnp.tanh(g4[:, 2 * f4:3 * f4])
    c_new = f_g * c4 + i_g * t_g
    o_g = jax.nn.sigmoid(g4[:, 3 * f4:4 * f4] + wco4 * c_new)
    h_new = o_g * jnp.tanh(c_new)
    c_new_vm[...] = c_new
    h_new_vm[...] = h_new
    row_id = jax.lax.broadcasted_iota(jnp.int32, (f4, 4), 0)
    col_id = jax.lax.broadcasted_iota(jnp.int32, (f4, 4), 1)
    lin_tile = jnp.concatenate([linw_ref[...]] * 4, axis=0)
    lin4 = jnp.where(row_id // F_OUT == col_id, lin_tile, 0.0)
    out_vm[...] = (jnp.dot(h_new, lin4, preferred_element_type=jnp.float32)
                   + linb_ref[...])
    cp_out = [pltpu.make_async_copy(out_vm, out_hbm, s3),
              pltpu.make_async_copy(h_new_vm, h_out_hbm, s4),
              pltpu.make_async_copy(c_new_vm, c_out_hbm, s5)]
    for cpy in cp_out:
        cpy.start()
    for cpy in cp_out:
        cpy.wait()


def kernel(x, edge_index, edge_weight, h, c, W_i, W_f, W_c, W_o, conv_i_w,
           conv_i_b, conv_f_w, conv_f_b, conv_c_w, conv_c_b, conv_o_w,
           conv_o_b, w_c_i, w_c_f, w_c_o, b_i, b_f, b_c, b_o, lin_w, lin_b):
    del edge_index, edge_weight
    n, f_in = x.shape
    f_out = h.shape[1]
    rows = n // 4

    x4 = x.reshape(rows, 4 * f_in)
    h4 = h.reshape(rows, 4 * f_out)
    c4 = c.reshape(rows, 4 * f_out)
    cib = conv_i_b.reshape(1, f_out)
    cfb = conv_f_b.reshape(1, f_out)
    ccb = conv_c_b.reshape(1, f_out)
    cob = conv_o_b.reshape(1, f_out)
    linb = lin_b.reshape(1, 1)

    any_spec = pl.BlockSpec(memory_space=pltpu.MemorySpace.HBM)
    full_spec = lambda a: pl.BlockSpec(a.shape, lambda: (0, 0))

    out4, h_new4, c_new4 = pl.pallas_call(
        _gclstm_block,
        in_specs=[
            any_spec, any_spec, any_spec,
            full_spec(W_i), full_spec(W_f), full_spec(W_c), full_spec(W_o),
            full_spec(conv_i_w), full_spec(conv_f_w), full_spec(conv_c_w),
            full_spec(conv_o_w),
            full_spec(cib), full_spec(cfb), full_spec(ccb), full_spec(cob),
            full_spec(w_c_i), full_spec(w_c_f), full_spec(w_c_o),
            full_spec(b_i), full_spec(b_f), full_spec(b_c), full_spec(b_o),
            full_spec(lin_w), full_spec(linb),
        ],
        out_specs=[any_spec, any_spec, any_spec],
        out_shape=[
            jax.ShapeDtypeStruct((rows, 4), jnp.float32),
            jax.ShapeDtypeStruct((rows, 4 * f_out), jnp.float32),
            jax.ShapeDtypeStruct((rows, 4 * f_out), jnp.float32),
        ],
        scratch_shapes=[
            pltpu.VMEM((rows, 4 * f_in), jnp.float32),
            pltpu.VMEM((rows, 4 * f_out), jnp.float32),
            pltpu.VMEM((rows, 4 * f_out), jnp.float32),
            pltpu.VMEM((rows, 4), jnp.float32),
            pltpu.VMEM((rows, 4 * f_out), jnp.float32),
            pltpu.VMEM((rows, 4 * f_out), jnp.float32),
        ] + [pltpu.SemaphoreType.DMA] * 6,
    )(x4, h4, c4, W_i, W_f, W_c, W_o, conv_i_w, conv_f_w, conv_c_w, conv_o_w,
      cib, cfb, ccb, cob, w_c_i, w_c_f, w_c_o, b_i, b_f, b_c, b_o,
      lin_w, linb)
    return (out4.reshape(n, 1), h_new4.reshape(n, f_out),
            c_new4.reshape(n, f_out))
